# baseline (speedup 1.0000x reference)
import functools

import jax
import jax.numpy as jnp
from jax import lax
from jax.experimental import pallas as pl
from jax.experimental.pallas import tpu as pltpu
from jax.experimental.pallas import tpu_sc as plsc


NC, NS, L = 2, 16, 16
NW = NC * NS

S, B, D, V = 200, 4096, 32, 1000000
BT = B // 128
UNITS = S * BT
PER_W = UNITS // NW
PAIRS = PER_W // 2


def _make_gather():
    mesh = plsc.VectorSubcoreMesh(core_axis_name="c", subcore_axis_name="s")

    def transpose_unit(rows, trans):
        iota = lax.iota(jnp.int32, L)
        bl_vecs = [iota + lg * L for lg in range(128 // L)]
        d_vecs = [jnp.full((L,), d, jnp.int32) for d in range(D)]
        for d in range(D):
            for lg in range(128 // L):
                vals = plsc.load_gather(rows, [bl_vecs[lg], d_vecs[d]])
                trans[d, pl.ds(lg * L, L)] = vals

    def body(ids_hbm, table_hbm, out_hbm, idx_a, idx_b, rows_a, rows_b,
             trans_a, trans_b, sem_g):
        wid = lax.axis_index("s") * NC + lax.axis_index("c")
        base = wid * PER_W
        idx_v = (idx_a, idx_b)
        rows_v = (rows_a, rows_b)
        trans_v = (trans_a, trans_b)

        def unit_coords(u):
            g = base + u
            s = g // BT
            bt = g - s * BT
            st = s // 8
            sl = s - st * 8
            return s, bt, st, sl

        def idx_copy(u, slot):
            _, bt, st, sl = unit_coords(u)
            pltpu.sync_copy(ids_hbm.at[st, bt, sl], idx_v[slot])

        def gather_start(slot):
            return pltpu.async_copy(table_hbm.at[idx_v[slot]],
                                    rows_v[slot], sem_g)

        def out_copy(u, slot):
            s, bt, _, _ = unit_coords(u)
            for dt in range(4):
                pltpu.sync_copy(trans_v[slot].at[pl.ds(dt * 8, 8)],
                                out_hbm.at[s, dt, bt])

        idx_copy(0, 0)
        gather_start(0)

        def pair(j, _):
            u = 2 * j
            idx_copy(u + 1, 1)
            pltpu.make_async_copy(table_hbm.at[idx_v[0]], rows_v[0],
                                  sem_g).wait()
            g1 = gather_start(1)
            transpose_unit(rows_v[0], trans_v[0])
            out_copy(u, 0)

            @pl.when(j + 1 < PAIRS)
            def _():
                idx_copy(u + 2, 0)
            g1.wait()

            @pl.when(j + 1 < PAIRS)
            def _():
                gather_start(0)
            transpose_unit(rows_v[1], trans_v[1])
            out_copy(u + 1, 1)
            return 0

        lax.fori_loop(0, PAIRS, pair, 0)

    return pl.kernel(
        body,
        out_type=jax.ShapeDtypeStruct((S, 4, BT, 8, 128), jnp.float32),
        mesh=mesh,
        scratch_types=[
            pltpu.VMEM((128,), jnp.int32),
            pltpu.VMEM((128,), jnp.int32),
            pltpu.VMEM((128, D), jnp.float32),
            pltpu.VMEM((128, D), jnp.float32),
            pltpu.VMEM((D, 128), jnp.float32),
            pltpu.VMEM((D, 128), jnp.float32),
            pltpu.SemaphoreType.DMA,
        ],
        compiler_params=pltpu.CompilerParams(use_tc_tiling_on_sc=False,
                                             needs_layout_passes=False),
    )


_GATHER = None


def kernel(input_ids, table):
    global _GATHER
    if _GATHER is None:
        _GATHER = _make_gather()
    ids4 = (input_ids.astype(jnp.int32).T
            .reshape(S // 8, 8, BT, 128).transpose(0, 2, 1, 3))
    out5 = _GATHER(ids4, table)
    return out5.transpose(2, 4, 0, 1, 3).reshape(B, S, D)


# speedup vs baseline: 1.5733x; 1.5733x over previous
import jax
import jax.numpy as jnp
from jax import lax
from jax.experimental import pallas as pl
from jax.experimental.pallas import tpu as pltpu
from jax.experimental.pallas import tpu_sc as plsc


NC, NS, L = 2, 16, 16
NW = NC * NS

S, B, D, V = 200, 4096, 32, 1000000
BT = B // 128
QT = BT // 4
UNITS = S * QT
PER_W = UNITS // NW
PAIRS = PER_W // 2


def _make_gather():
    mesh = plsc.VectorSubcoreMesh(core_axis_name="c", subcore_axis_name="s")

    def transpose_unit(rows, trans):
        iota = lax.iota(jnp.int32, L)
        bl_vecs = [iota + lg * L for lg in range(128 // L)]
        for qq in range(4):
            for lg in range(128 // L):
                tok_vec = bl_vecs[lg] + qq * 128
                prev = None
                for g in range(4):
                    vals = [
                        plsc.load_gather(
                            rows,
                            [tok_vec,
                             jnp.full((L,), 8 * g + k, jnp.int32)])
                        for k in range(8)
                    ]
                    if prev is not None:
                        for k in range(8):
                            trans[g - 1, qq, k, pl.ds(lg * L, L)] = prev[k]
                    prev = vals
                for k in range(8):
                    trans[3, qq, k, pl.ds(lg * L, L)] = prev[k]

    def body(ids_hbm, table_hbm, out_hbm, idx_a, idx_b, rows_a, rows_b,
             trans_a, trans_b, sem_i, sem_g, sem_o):
        wid = lax.axis_index("s") * NC + lax.axis_index("c")
        base = wid * PER_W
        idx_v = (idx_a, idx_b)
        rows_v = (rows_a, rows_b)
        trans_v = (trans_a, trans_b)

        def unit_coords(u):
            g = base + u
            s = g // QT
            qt = g - s * QT
            st = s // 8
            sl = s - st * 8
            return s, qt, st, sl

        def idx_fire(u, slot):
            _, qt, st, sl = unit_coords(u)
            for q in range(4):
                pltpu.async_copy(ids_hbm.at[st, 4 * qt + q, sl],
                                 idx_v[slot].at[pl.ds(128 * q, 128)],
                                 sem_i)

        def idx_wait(slot):
            for q in range(4):
                pltpu.make_async_copy(ids_hbm.at[0, 0, 0],
                                      idx_v[slot].at[pl.ds(128 * q, 128)],
                                      sem_i).wait()

        def gather_fire(slot):
            pltpu.async_copy(table_hbm.at[idx_v[slot]], rows_v[slot], sem_g)

        def gather_wait(slot):
            pltpu.make_async_copy(table_hbm.at[idx_v[slot]], rows_v[slot],
                                  sem_g).wait()

        def out_fire(u, slot):
            s, qt, _, _ = unit_coords(u)
            pltpu.async_copy(trans_v[slot],
                             out_hbm.at[s, :, pl.ds(4 * qt, 4)], sem_o)

        def out_drain(slot):
            pltpu.make_async_copy(trans_v[slot],
                                  out_hbm.at[0, :, pl.ds(0, 4)],
                                  sem_o).wait()

        idx_fire(0, 0)
        idx_wait(0)
        gather_fire(0)
        idx_fire(1, 1)

        def pair(j, _):
            a = 2 * j

            @pl.when(j > 0)
            def _():
                out_drain(0)
            gather_wait(0)
            idx_wait(1)
            gather_fire(1)

            @pl.when(j + 1 < PAIRS)
            def _():
                idx_fire(a + 2, 0)
            transpose_unit(rows_v[0], trans_v[0])
            out_fire(a, 0)

            @pl.when(j > 0)
            def _():
                out_drain(1)
            gather_wait(1)

            @pl.when(j + 1 < PAIRS)
            def _():
                idx_wait(0)
                gather_fire(0)
                idx_fire(a + 3, 1)
            transpose_unit(rows_v[1], trans_v[1])
            out_fire(a + 1, 1)
            return 0

        lax.fori_loop(0, PAIRS, pair, 0)
        out_drain(0)
        out_drain(1)

    return pl.kernel(
        body,
        out_type=jax.ShapeDtypeStruct((S, 4, BT, 8, 128), jnp.float32),
        mesh=mesh,
        scratch_types=[
            pltpu.VMEM((512,), jnp.int32),
            pltpu.VMEM((512,), jnp.int32),
            pltpu.VMEM((512, D), jnp.float32),
            pltpu.VMEM((512, D), jnp.float32),
            pltpu.VMEM((4, 4, 8, 128), jnp.float32),
            pltpu.VMEM((4, 4, 8, 128), jnp.float32),
            pltpu.SemaphoreType.DMA,
            pltpu.SemaphoreType.DMA,
            pltpu.SemaphoreType.DMA,
        ],
        compiler_params=pltpu.CompilerParams(use_tc_tiling_on_sc=False,
                                             needs_layout_passes=False),
    )


_GATHER = None


def kernel(input_ids, table):
    global _GATHER
    if _GATHER is None:
        _GATHER = _make_gather()
    ids4 = (input_ids.astype(jnp.int32).T
            .reshape(S // 8, 8, BT, 128).transpose(0, 2, 1, 3))
    out5 = _GATHER(ids4, table)
    return out5.transpose(2, 4, 0, 1, 3).reshape(B, S, D)
